# baseline (speedup 1.0000x reference)
import functools

import jax
import jax.numpy as jnp
from jax import lax
from jax.experimental import pallas as pl
from jax.experimental.pallas import tpu as pltpu
from jax.experimental.pallas import tpu_sc as plsc

N = 10000
NP = 10240
EB = 128
NW = 32
RT = NP // 16
DEGW = 16

_f32 = jnp.float32


def _make_seg(D, rt, ch, eb, nb, with_deg):
    nchunk = rt // ch
    nround = ch // nb
    mesh = plsc.VectorSubcoreMesh(core_axis_name="c", subcore_axis_name="s")
    out_type = [jax.ShapeDtypeStruct((2, NP, D), _f32)]
    scratch = [
        pltpu.VMEM((ch, eb), jnp.int32),
        pltpu.VMEM((ch, eb), jnp.int32),
    ]
    scratch += [pltpu.VMEM((eb, D), _f32) for _ in range(nb)]
    scratch += [pltpu.VMEM_SHARED((NP, D), _f32)]
    scratch += [pltpu.SemaphoreType.DMA for _ in range(2 * nb)]
    if with_deg:
        out_type.append(jax.ShapeDtypeStruct((2, NP, DEGW), _f32))
        scratch += [
            pltpu.VMEM((eb, DEGW), _f32),
            pltpu.VMEM_SHARED((NP, DEGW), _f32),
        ]
        scratch += [pltpu.SemaphoreType.DMA for _ in range(nb)]

    def body(table, srci, dsti, zeros_f, *rest):
        if with_deg:
            (zeros_d, ones_h, aggp, degp, src_v, dst_v, *more) = rest
        else:
            (aggp, src_v, dst_v, *more) = rest
        fb = more[:nb]
        acc = more[nb]
        sem_g = more[nb + 1:2 * nb + 1]
        sem_s = more[2 * nb + 1:3 * nb + 1]
        if with_deg:
            ones_v, dacc = more[3 * nb + 1:3 * nb + 3]
            sem_d = more[3 * nb + 3:]
        c = lax.axis_index("c")
        s = lax.axis_index("s")
        pltpu.sync_copy(zeros_f.at[pl.ds(s * RT, RT)], acc.at[pl.ds(s * RT, RT)])
        if with_deg:
            pltpu.sync_copy(zeros_d.at[pl.ds(s * RT, RT)],
                            dacc.at[pl.ds(s * RT, RT)])
            pltpu.sync_copy(ones_h, ones_v)
        plsc.subcore_barrier()

        base = (c * 16 + s) * rt

        def wait_and_scatter(j, b):
            pltpu.make_async_copy(table.at[src_v.at[j]], fb[b], sem_g[b]).wait()
            pltpu.async_copy(fb[b], acc.at[dst_v.at[j]], sem_s[b], add=True)
            if with_deg:
                pltpu.async_copy(ones_v, dacc.at[dst_v.at[j]], sem_d[b],
                                 add=True)

        def drain(j, b):
            pltpu.make_async_copy(fb[b], acc.at[dst_v.at[j]], sem_s[b]).wait()
            if with_deg:
                pltpu.make_async_copy(ones_v, dacc.at[dst_v.at[j]],
                                      sem_d[b]).wait()

        def chunk_body(cc, carry):
            row0 = base + cc * ch
            pltpu.sync_copy(srci.at[pl.ds(row0, ch)], src_v)
            pltpu.sync_copy(dsti.at[pl.ds(row0, ch)], dst_v)
            for b in range(nb):
                pltpu.async_copy(table.at[src_v.at[b]], fb[b], sem_g[b])

            def round_body(r, carry2):
                j0 = r * nb
                for b in range(nb):
                    wait_and_scatter(j0 + b, b)
                for b in range(nb):
                    drain(j0 + b, b)
                    pltpu.async_copy(table.at[src_v.at[j0 + nb + b]],
                                     fb[b], sem_g[b])
                return carry2

            lax.fori_loop(0, nround - 1, round_body, 0)
            j0 = (nround - 1) * nb
            for b in range(nb):
                wait_and_scatter(j0 + b, b)
            for b in range(nb):
                drain(j0 + b, b)
            return carry

        lax.fori_loop(0, nchunk, chunk_body, 0)
        plsc.subcore_barrier()
        pltpu.sync_copy(acc.at[pl.ds(s * RT, RT)], aggp.at[c, pl.ds(s * RT, RT)])
        if with_deg:
            pltpu.sync_copy(dacc.at[pl.ds(s * RT, RT)],
                            degp.at[c, pl.ds(s * RT, RT)])

    return pl.kernel(body, out_type=out_type, mesh=mesh, scratch_types=scratch,
                     compiler_params=pltpu.CompilerParams(
                         use_tc_tiling_on_sc=False))


_GRID = 8
_BR = NP // _GRID


def _rows(i):
    return (i, 0)


def _rows3(i):
    return (0, i, 0)


def _full(i):
    return (0, 0)


def _deg_of(degp_ref):
    return degp_ref[0, :, 0:1] + degp_ref[1, :, 0:1]


def _stage2_body(aggp_ref, degp_ref, x_ref, wl1t_ref, wr1t_ref, bl1_ref,
                 wl2t_ref, wr2t_ref, hl2_ref, hr2_ref):
    deg = _deg_of(degp_ref)
    mean = (aggp_ref[0] + aggp_ref[1]) / jnp.maximum(deg, 1.0)
    h = jnp.dot(mean, wl1t_ref[...], preferred_element_type=_f32)
    h += jnp.dot(x_ref[...], wr1t_ref[...], preferred_element_type=_f32)
    h = jnp.maximum(h + bl1_ref[...], 0.0)
    hl2_ref[...] = jnp.dot(h, wl2t_ref[...], preferred_element_type=_f32)
    hr2_ref[...] = jnp.dot(h, wr2t_ref[...], preferred_element_type=_f32)


def _stage2(agg1p, degp, x, wl1t, wr1t, bl1, wl2t, wr2t, n):
    br = n // 10
    return pl.pallas_call(
        _stage2_body,
        grid=(10,),
        in_specs=[
            pl.BlockSpec((2, br, 128), _rows3),
            pl.BlockSpec((2, br, DEGW), _rows3),
            pl.BlockSpec((br, 128), _rows),
            pl.BlockSpec((128, 128), _full),
            pl.BlockSpec((128, 128), _full),
            pl.BlockSpec((1, 128), _full),
            pl.BlockSpec((128, 40), _full),
            pl.BlockSpec((128, 40), _full),
        ],
        out_specs=[pl.BlockSpec((br, 40), _rows)] * 2,
        out_shape=[jax.ShapeDtypeStruct((n, 40), _f32)] * 2,
    )(agg1p, degp, x, wl1t, wr1t, bl1, wl2t, wr2t)


def _stage3_body(aggp_ref, degp_ref, hr2_ref, wgt_ref, bl2_ref, y_ref):
    deg = _deg_of(degp_ref)
    mean = (aggp_ref[0] + aggp_ref[1]) / jnp.maximum(deg, 1.0)
    h2 = jnp.maximum(mean + bl2_ref[...] + hr2_ref[...], 0.0)
    dinv = lax.rsqrt(deg + 1.0)
    y_ref[...] = jnp.dot(h2, wgt_ref[...], preferred_element_type=_f32) * dinv


def _stage3(agg2p, degp, hr2, wgt, bl2, n):
    br = n // 10
    return pl.pallas_call(
        _stage3_body,
        grid=(10,),
        in_specs=[
            pl.BlockSpec((2, br, 40), _rows3),
            pl.BlockSpec((2, br, DEGW), _rows3),
            pl.BlockSpec((br, 40), _rows),
            pl.BlockSpec((40, 40), _full),
            pl.BlockSpec((1, 40), _full),
        ],
        out_specs=pl.BlockSpec((br, 40), _rows),
        out_shape=jax.ShapeDtypeStruct((n, 40), _f32),
    )(agg2p, degp, hr2, wgt, bl2)


def _stage4_body(aggp_ref, degp_ref, y_ref, bg_ref, o_ref):
    deg = _deg_of(degp_ref)
    dinv = lax.rsqrt(deg + 1.0)
    o_ref[...] = dinv * (aggp_ref[0] + aggp_ref[1] + y_ref[...]) + bg_ref[...]


def _stage4(agggp, degp, y, bg, n):
    br = n // 10
    return pl.pallas_call(
        _stage4_body,
        grid=(10,),
        in_specs=[
            pl.BlockSpec((2, br, 40), _rows3),
            pl.BlockSpec((2, br, DEGW), _rows3),
            pl.BlockSpec((br, 40), _rows),
            pl.BlockSpec((1, 40), _full),
        ],
        out_specs=pl.BlockSpec((br, 40), _rows),
        out_shape=jax.ShapeDtypeStruct((n, 40), _f32),
    )(agggp, degp, y, bg)


def kernel(x, edge_index, Wl1, bl1, Wr1, Wl2, bl2, Wr2, Wg, bg):
    n, f_in = x.shape
    e = edge_index.shape[1]
    er = -(-e // EB)
    er_pad = -(-er // (NW * 8)) * (NW * 8)
    rw = er_pad // NW
    pad_e = er_pad * EB - e

    lane = jnp.arange(pad_e, dtype=jnp.int32) % EB
    src = jnp.concatenate([edge_index[0], lane]).reshape(er_pad, EB)
    dst = jnp.concatenate([edge_index[1], lane + n]).reshape(er_pad, EB)

    zeros128 = jnp.zeros((NP, 128), _f32)
    zeros40 = jnp.zeros((NP, 40), _f32)
    zeros_d = jnp.zeros((NP, DEGW), _f32)
    ones_h = jnp.ones((EB, DEGW), _f32)

    seg128 = _make_seg(128, rw, rw // 10, EB, 2, with_deg=True)
    seg40 = _make_seg(40, rw, rw, EB, 8, with_deg=False)

    agg1p, degp = seg128(x, src, dst, zeros128, zeros_d, ones_h)
    hl2, hr2 = _stage2(agg1p, degp, x, Wl1.T, Wr1.T, bl1.reshape(1, 128),
                       Wl2.T, Wr2.T, n)
    (agg2p,) = seg40(hl2, src, dst, zeros40)
    y = _stage3(agg2p, degp, hr2, Wg.T, bl2.reshape(1, 40), n)
    (agggp,) = seg40(y, src, dst, zeros40)
    return _stage4(agggp, degp, y, bg.reshape(1, 40), n)


# speedup vs baseline: 1.0620x; 1.0620x over previous
import functools

import jax
import jax.numpy as jnp
from jax import lax
from jax.experimental import pallas as pl
from jax.experimental.pallas import tpu as pltpu
from jax.experimental.pallas import tpu_sc as plsc

N = 10000
NP = 10240
EB = 128
NW = 32
RT = NP // 16
DEGW = 16

_f32 = jnp.float32


def _make_seg(D, rt, ch, eb, nb, with_deg):
    nchunk = rt // ch
    nround = ch // nb
    mesh = plsc.VectorSubcoreMesh(core_axis_name="c", subcore_axis_name="s")
    out_type = [jax.ShapeDtypeStruct((2, NP, D), _f32)]
    scratch = [
        pltpu.VMEM((ch, eb), jnp.int32),
        pltpu.VMEM((ch, eb), jnp.int32),
    ]
    scratch += [pltpu.VMEM((eb, D), _f32) for _ in range(nb)]
    scratch += [pltpu.VMEM_SHARED((NP, D), _f32)]
    scratch += [pltpu.SemaphoreType.DMA for _ in range(2 * nb)]
    if with_deg:
        out_type.append(jax.ShapeDtypeStruct((2, NP, DEGW), _f32))
        scratch += [
            pltpu.VMEM((eb, DEGW), _f32),
            pltpu.VMEM_SHARED((NP, DEGW), _f32),
        ]
        scratch += [pltpu.SemaphoreType.DMA for _ in range(nb)]

    def body(table, srci, dsti, zeros_f, *rest):
        if with_deg:
            (zeros_d, ones_h, aggp, degp, src_v, dst_v, *more) = rest
        else:
            (aggp, src_v, dst_v, *more) = rest
        fb = more[:nb]
        acc = more[nb]
        sem_g = more[nb + 1:2 * nb + 1]
        sem_s = more[2 * nb + 1:3 * nb + 1]
        if with_deg:
            ones_v, dacc = more[3 * nb + 1:3 * nb + 3]
            sem_d = more[3 * nb + 3:]
        c = lax.axis_index("c")
        s = lax.axis_index("s")
        pltpu.sync_copy(zeros_f.at[pl.ds(s * RT, RT)], acc.at[pl.ds(s * RT, RT)])
        if with_deg:
            pltpu.sync_copy(zeros_d.at[pl.ds(s * RT, RT)],
                            dacc.at[pl.ds(s * RT, RT)])
            pltpu.sync_copy(ones_h, ones_v)
        plsc.subcore_barrier()

        base = (c * 16 + s) * rt

        def wait_and_scatter(j, b):
            pltpu.make_async_copy(table.at[src_v.at[j]], fb[b], sem_g[b]).wait()
            pltpu.async_copy(fb[b], acc.at[dst_v.at[j]], sem_s[b], add=True)
            if with_deg:
                pltpu.async_copy(ones_v, dacc.at[dst_v.at[j]], sem_d[b],
                                 add=True)

        def drain(j, b):
            pltpu.make_async_copy(fb[b], acc.at[dst_v.at[j]], sem_s[b]).wait()
            if with_deg:
                pltpu.make_async_copy(ones_v, dacc.at[dst_v.at[j]],
                                      sem_d[b]).wait()

        def chunk_body(cc, carry):
            row0 = base + cc * ch
            pltpu.sync_copy(srci.at[pl.ds(row0, ch)], src_v)
            pltpu.sync_copy(dsti.at[pl.ds(row0, ch)], dst_v)
            for b in range(nb):
                pltpu.async_copy(table.at[src_v.at[b]], fb[b], sem_g[b])

            def round_body(r, carry2):
                j0 = r * nb
                for b in range(nb):
                    wait_and_scatter(j0 + b, b)
                for b in range(nb):
                    drain(j0 + b, b)
                    pltpu.async_copy(table.at[src_v.at[j0 + nb + b]],
                                     fb[b], sem_g[b])
                return carry2

            lax.fori_loop(0, nround - 1, round_body, 0)
            j0 = (nround - 1) * nb
            for b in range(nb):
                wait_and_scatter(j0 + b, b)
            for b in range(nb):
                drain(j0 + b, b)
            return carry

        lax.fori_loop(0, nchunk, chunk_body, 0)
        plsc.subcore_barrier()
        pltpu.sync_copy(acc.at[pl.ds(s * RT, RT)], aggp.at[c, pl.ds(s * RT, RT)])
        if with_deg:
            pltpu.sync_copy(dacc.at[pl.ds(s * RT, RT)],
                            degp.at[c, pl.ds(s * RT, RT)])

    return pl.kernel(body, out_type=out_type, mesh=mesh, scratch_types=scratch,
                     compiler_params=pltpu.CompilerParams(
                         use_tc_tiling_on_sc=False))


_GRID = 8
_BR = NP // _GRID


def _rows(i):
    return (i, 0)


def _rows3(i):
    return (0, i, 0)


def _full(i):
    return (0, 0)


def _deg_of(degp_ref):
    return degp_ref[0, :, 0:1] + degp_ref[1, :, 0:1]


def _stage2_body(aggp_ref, degp_ref, x_ref, wl1t_ref, wr1t_ref, bl1_ref,
                 wl2t_ref, wr2t_ref, hl2_ref, hr2_ref):
    deg = _deg_of(degp_ref)
    mean = (aggp_ref[0] + aggp_ref[1]) / jnp.maximum(deg, 1.0)
    h = jnp.dot(mean, wl1t_ref[...], preferred_element_type=_f32)
    h += jnp.dot(x_ref[...], wr1t_ref[...], preferred_element_type=_f32)
    h = jnp.maximum(h + bl1_ref[...], 0.0)
    hl2_ref[...] = jnp.dot(h, wl2t_ref[...], preferred_element_type=_f32)
    hr2_ref[...] = jnp.dot(h, wr2t_ref[...], preferred_element_type=_f32)


def _stage2(agg1p, degp, x, wl1t, wr1t, bl1, wl2t, wr2t, n):
    br = n // 10
    return pl.pallas_call(
        _stage2_body,
        grid=(10,),
        in_specs=[
            pl.BlockSpec((2, br, 128), _rows3),
            pl.BlockSpec((2, br, DEGW), _rows3),
            pl.BlockSpec((br, 128), _rows),
            pl.BlockSpec((128, 128), _full),
            pl.BlockSpec((128, 128), _full),
            pl.BlockSpec((1, 128), _full),
            pl.BlockSpec((128, 40), _full),
            pl.BlockSpec((128, 40), _full),
        ],
        out_specs=[pl.BlockSpec((br, 40), _rows)] * 2,
        out_shape=[jax.ShapeDtypeStruct((n, 40), _f32)] * 2,
    )(agg1p, degp, x, wl1t, wr1t, bl1, wl2t, wr2t)


def _stage3_body(aggp_ref, degp_ref, hr2_ref, wgt_ref, bl2_ref, y_ref):
    deg = _deg_of(degp_ref)
    mean = (aggp_ref[0] + aggp_ref[1]) / jnp.maximum(deg, 1.0)
    h2 = jnp.maximum(mean + bl2_ref[...] + hr2_ref[...], 0.0)
    dinv = lax.rsqrt(deg + 1.0)
    y_ref[...] = jnp.dot(h2, wgt_ref[...], preferred_element_type=_f32) * dinv


def _stage3(agg2p, degp, hr2, wgt, bl2, n):
    br = n // 10
    return pl.pallas_call(
        _stage3_body,
        grid=(10,),
        in_specs=[
            pl.BlockSpec((2, br, 40), _rows3),
            pl.BlockSpec((2, br, DEGW), _rows3),
            pl.BlockSpec((br, 40), _rows),
            pl.BlockSpec((40, 40), _full),
            pl.BlockSpec((1, 40), _full),
        ],
        out_specs=pl.BlockSpec((br, 40), _rows),
        out_shape=jax.ShapeDtypeStruct((n, 40), _f32),
    )(agg2p, degp, hr2, wgt, bl2)


def _stage4_body(aggp_ref, degp_ref, y_ref, bg_ref, o_ref):
    deg = _deg_of(degp_ref)
    dinv = lax.rsqrt(deg + 1.0)
    o_ref[...] = dinv * (aggp_ref[0] + aggp_ref[1] + y_ref[...]) + bg_ref[...]


def _stage4(agggp, degp, y, bg, n):
    br = n // 10
    return pl.pallas_call(
        _stage4_body,
        grid=(10,),
        in_specs=[
            pl.BlockSpec((2, br, 40), _rows3),
            pl.BlockSpec((2, br, DEGW), _rows3),
            pl.BlockSpec((br, 40), _rows),
            pl.BlockSpec((1, 40), _full),
        ],
        out_specs=pl.BlockSpec((br, 40), _rows),
        out_shape=jax.ShapeDtypeStruct((n, 40), _f32),
    )(agggp, degp, y, bg)


def kernel(x, edge_index, Wl1, bl1, Wr1, Wl2, bl2, Wr2, Wg, bg):
    n, f_in = x.shape
    e = edge_index.shape[1]
    er = -(-e // EB)
    er_pad = -(-er // (NW * 8)) * (NW * 8)
    rw = er_pad // NW
    pad_e = er_pad * EB - e

    lane = jnp.arange(pad_e, dtype=jnp.int32) % EB
    src = jnp.concatenate([edge_index[0], lane]).reshape(er_pad, EB)
    dst = jnp.concatenate([edge_index[1], lane + n]).reshape(er_pad, EB)

    zeros128 = jnp.zeros((NP, 128), _f32)
    zeros40 = jnp.zeros((NP, 40), _f32)
    zeros_d = jnp.zeros((NP, DEGW), _f32)
    ones_h32 = jnp.ones((32, DEGW), _f32)

    src32 = src.reshape(er_pad * 4, 32)
    dst32 = dst.reshape(er_pad * 4, 32)
    rt1 = (er_pad * 4) // NW
    seg128 = _make_seg(128, rt1, rt1 // 5, 32, 8, with_deg=True)
    seg40 = _make_seg(40, rw, rw, EB, 8, with_deg=False)

    agg1p, degp = seg128(x, src32, dst32, zeros128, zeros_d, ones_h32)
    hl2, hr2 = _stage2(agg1p, degp, x, Wl1.T, Wr1.T, bl1.reshape(1, 128),
                       Wl2.T, Wr2.T, n)
    (agg2p,) = seg40(hl2, src, dst, zeros40)
    y = _stage3(agg2p, degp, hr2, Wg.T, bl2.reshape(1, 40), n)
    (agggp,) = seg40(y, src, dst, zeros40)
    return _stage4(agggp, degp, y, bg.reshape(1, 40), n)
